# baseline (speedup 1.0000x reference)
import functools

import jax
import jax.numpy as jnp
from jax import lax
from jax.experimental import pallas as pl
from jax.experimental.pallas import tpu as pltpu
from jax.experimental.pallas import tpu_sc as plsc

_VOCAB = 1000000
_EMBED_DIM = 64
_BATCH = 4096
_HIST = 200
_B = _BATCH * _HIST

_NC = 2
_NS = 16
_NW = _NC * _NS
_PER_W = _B // _NW
_CHUNK = 128
_NCHUNKS = _PER_W // _CHUNK


def _build():
    mesh = plsc.VectorSubcoreMesh(core_axis_name="c", subcore_axis_name="s")

    @functools.partial(
        pl.kernel,
        out_type=jax.ShapeDtypeStruct((_B, _EMBED_DIM), jnp.float32),
        mesh=mesh,
        scratch_types=[
            pltpu.VMEM((_CHUNK,), jnp.int32),
            pltpu.VMEM((_CHUNK, _EMBED_DIM), jnp.float32),
            pltpu.SemaphoreType.DMA,
        ],
        compiler_params=pltpu.CompilerParams(use_tc_tiling_on_sc=False),
    )
    def gather_kernel(table_hbm, idx_hbm, out_hbm, idx_v, rows_v, sem):
        wid = lax.axis_index("s") * _NC + lax.axis_index("c")
        base = wid * _PER_W

        def chunk(i, carry):
            off = base + i * _CHUNK
            pltpu.sync_copy(idx_hbm.at[pl.ds(off, _CHUNK)], idx_v)
            pltpu.async_copy(table_hbm.at[idx_v], rows_v, sem).wait()
            pltpu.sync_copy(rows_v, out_hbm.at[pl.ds(off, _CHUNK)])
            return carry

        lax.fori_loop(0, _NCHUNKS, chunk, 0)

    return gather_kernel


_gather = _build()


def kernel(inputs, W):
    idx = inputs.reshape(-1).astype(jnp.int32)
    out = _gather(W, idx)
    return out.reshape(_BATCH, _HIST, _EMBED_DIM)


# speedup vs baseline: 1.1910x; 1.1910x over previous
import functools

import jax
import jax.numpy as jnp
from jax import lax
from jax.experimental import pallas as pl
from jax.experimental.pallas import tpu as pltpu
from jax.experimental.pallas import tpu_sc as plsc

_VOCAB = 1000000
_EMBED_DIM = 64
_BATCH = 4096
_HIST = 200
_B = _BATCH * _HIST

_NC = 2
_NS = 16
_NW = _NC * _NS
_PER_W = _B // _NW
_CHUNK = 128
_G = 5
_GROWS = _G * _CHUNK
_NG = _PER_W // _GROWS


def _build():
    mesh = plsc.VectorSubcoreMesh(core_axis_name="c", subcore_axis_name="s")

    @functools.partial(
        pl.kernel,
        out_type=jax.ShapeDtypeStruct((_B, _EMBED_DIM), jnp.float32),
        mesh=mesh,
        scratch_types=[
            pltpu.VMEM((_PER_W,), jnp.int32),
            pltpu.VMEM((_GROWS, _EMBED_DIM), jnp.float32),
            pltpu.VMEM((_GROWS, _EMBED_DIM), jnp.float32),
            pltpu.SemaphoreType.DMA,
            pltpu.SemaphoreType.DMA,
            pltpu.SemaphoreType.DMA,
            pltpu.SemaphoreType.DMA,
        ],
        compiler_params=pltpu.CompilerParams(use_tc_tiling_on_sc=False),
    )
    def gather_kernel(table_hbm, idx_hbm, out_hbm, idx_v, buf0, buf1,
                      g0, g1, s0, s1):
        wid = lax.axis_index("s") * _NC + lax.axis_index("c")
        base = wid * _PER_W
        pltpu.sync_copy(idx_hbm.at[pl.ds(base, _PER_W)], idx_v)

        def fire_gathers(gi, buf, gsem):
            for j in range(_G):
                off = gi * _GROWS + j * _CHUNK
                pltpu.async_copy(
                    table_hbm.at[idx_v.at[pl.ds(off, _CHUNK)]],
                    buf.at[pl.ds(j * _CHUNK, _CHUNK)],
                    gsem)

        def drain_gathers(buf, gsem):
            pltpu.make_async_copy(
                out_hbm.at[pl.ds(base, _GROWS)], buf, gsem).wait()

        def start_store(gi, buf, ssem):
            pltpu.async_copy(
                buf, out_hbm.at[pl.ds(base + gi * _GROWS, _GROWS)], ssem)

        def drain_store(gi, buf, ssem):
            pltpu.make_async_copy(
                buf, out_hbm.at[pl.ds(base + gi * _GROWS, _GROWS)],
                ssem).wait()

        fire_gathers(0, buf0, g0)

        def body_k(k, carry):
            a = 2 * k
            b = a + 1
            drain_gathers(buf0, g0)
            start_store(a, buf0, s0)

            @pl.when(k > 0)
            def _():
                drain_store(a - 1, buf1, s1)

            fire_gathers(b, buf1, g1)
            drain_gathers(buf1, g1)
            start_store(b, buf1, s1)
            drain_store(a, buf0, s0)

            @pl.when(k < _NG // 2 - 1)
            def _():
                fire_gathers(b + 1, buf0, g0)

            return carry

        lax.fori_loop(0, _NG // 2, body_k, 0)
        drain_store(_NG - 1, buf1, s1)

    return gather_kernel


_gather = _build()


def kernel(inputs, W):
    idx = inputs.reshape(-1).astype(jnp.int32)
    out = _gather(W, idx)
    return out.reshape(_BATCH, _HIST, _EMBED_DIM)
